# Initial kernel scaffold; baseline (speedup 1.0000x reference)
import jax
import jax.numpy as jnp
from jax.experimental import pallas as pl


def kernel(phrase, phrase_emb_weight, pos_emb_weight):
    raise NotImplementedError("write your pallas kernel here")


# speedup vs baseline: 6.2217x; 6.2217x over previous
import functools

import jax
import jax.numpy as jnp
from jax import lax
from jax.experimental import pallas as pl
from jax.experimental.pallas import tpu as pltpu
from jax.experimental.pallas import tpu_sc as plsc

D = 64
L = 50
NC = 2
NS = 16
NW = NC * NS
GB = 100
CHUNK = 200


def _build_kernel(n_chunks):
    mesh = plsc.VectorSubcoreMesh(core_axis_name="c", subcore_axis_name="s")
    rows_per_w = n_chunks * CHUNK
    total_rows = NW * rows_per_w

    @functools.partial(
        pl.kernel,
        mesh=mesh,
        compiler_params=pltpu.CompilerParams(use_tc_tiling_on_sc=False),
        out_type=jax.ShapeDtypeStruct((total_rows, D), jnp.float32),
        scratch_types=[
            pltpu.VMEM((n_chunks, CHUNK // GB, GB), jnp.int32),
            pltpu.VMEM((L, D), jnp.float32),
            pltpu.VMEM((CHUNK, D), jnp.float32),
            pltpu.SemaphoreType.DMA,
        ],
    )
    def gather_add(idx_hbm, table_hbm, pos_hbm, out_hbm, idx_v, pos_v, rows_v, sem):
        wid = lax.axis_index("s") * NC + lax.axis_index("c")
        base = wid * rows_per_w
        pltpu.sync_copy(idx_hbm.at[wid], idx_v)
        pltpu.sync_copy(pos_hbm, pos_v)

        def chunk_body(ci, carry):
            cps = [
                pltpu.async_copy(
                    table_hbm.at[idx_v.at[ci, g]],
                    rows_v.at[pl.ds(g * GB, GB)],
                    sem,
                )
                for g in range(CHUNK // GB)
            ]
            for cp in cps:
                cp.wait()
            for l in range(L):
                for j in range(D // 16):
                    pv = pos_v[l, pl.ds(j * 16, 16)]
                    for ph in range(CHUNK // L):
                        plsc.addupdate(rows_v.at[ph * L + l, pl.ds(j * 16, 16)], pv)
            pltpu.sync_copy(rows_v, out_hbm.at[pl.ds(base + ci * CHUNK, CHUNK)])
            return carry

        lax.fori_loop(0, n_chunks, chunk_body, 0)

    return gather_add


def kernel(phrase, phrase_emb_weight, pos_emb_weight):
    B, Lseq = phrase.shape
    rows = B * Lseq
    n_chunks = rows // (NW * CHUNK)
    idx = phrase.reshape(NW, n_chunks, CHUNK // GB, GB).astype(jnp.int32)
    pos = pos_emb_weight[:Lseq]
    out = _build_kernel(n_chunks)(idx, phrase_emb_weight, pos)
    return out.reshape(B, Lseq, phrase_emb_weight.shape[1])


# speedup vs baseline: 6.7406x; 1.0834x over previous
import functools

import jax
import jax.numpy as jnp
from jax import lax
from jax.experimental import pallas as pl
from jax.experimental.pallas import tpu as pltpu
from jax.experimental.pallas import tpu_sc as plsc

D = 64
L = 50
NC = 2
NS = 16
NW = NC * NS
GB = 100
CHUNK = 200
NBUF = 4


def _build_kernel(n_chunks):
    mesh = plsc.VectorSubcoreMesh(core_axis_name="c", subcore_axis_name="s")
    rows_per_w = n_chunks * CHUNK
    total_rows = NW * rows_per_w
    assert n_chunks % NBUF == 0

    @functools.partial(
        pl.kernel,
        mesh=mesh,
        compiler_params=pltpu.CompilerParams(use_tc_tiling_on_sc=False),
        out_type=jax.ShapeDtypeStruct((total_rows, D), jnp.float32),
        scratch_types=(
            [pltpu.VMEM((n_chunks, CHUNK // GB, GB), jnp.int32),
             pltpu.VMEM((L, D), jnp.float32)]
            + [pltpu.VMEM((CHUNK, D), jnp.float32) for _ in range(NBUF)]
            + [pltpu.SemaphoreType.DMA for _ in range(2 * NBUF)]
        ),
    )
    def gather_add(idx_hbm, table_hbm, pos_hbm, out_hbm, idx_v, pos_v, *rest):
        bufs = rest[:NBUF]
        gsem = rest[NBUF:2 * NBUF]
        wsem = rest[2 * NBUF:]
        wid = lax.axis_index("s") * NC + lax.axis_index("c")
        base = wid * rows_per_w
        pltpu.sync_copy(idx_hbm.at[wid], idx_v)
        pltpu.sync_copy(pos_hbm, pos_v)

        def start_gather(ci, b):
            for g in range(CHUNK // GB):
                pltpu.async_copy(
                    table_hbm.at[idx_v.at[ci, g]],
                    bufs[b].at[pl.ds(g * GB, GB)],
                    gsem[b],
                )

        def drain_gather(b):
            for g in range(CHUNK // GB):
                pltpu.make_async_copy(
                    table_hbm.at[idx_v.at[0, g]],
                    bufs[b].at[pl.ds(g * GB, GB)],
                    gsem[b],
                ).wait()

        def add_pos(b):
            def phrase_body(ph, carry):
                row = ph * L
                for l in range(L):
                    for j in range(D // 16):
                        pv = pos_v[l, pl.ds(j * 16, 16)]
                        plsc.addupdate(bufs[b].at[row + l, pl.ds(j * 16, 16)], pv)
                return carry
            lax.fori_loop(0, CHUNK // L, phrase_body, 0)

        def start_wb(ci, b):
            pltpu.async_copy(
                bufs[b], out_hbm.at[pl.ds(base + ci * CHUNK, CHUNK)], wsem[b])

        def drain_wb(ci, b):
            pltpu.make_async_copy(
                bufs[b], out_hbm.at[pl.ds(base + ci * CHUNK, CHUNK)], wsem[b]).wait()

        start_gather(0, 0)
        start_gather(1, 1)

        def outer(go, carry):
            for b in range(NBUF):
                ci = go * NBUF + b
                hb = (b + 2) % NBUF
                hi = ci + 2
                @pl.when(ci >= 2)
                def _():
                    drain_wb(ci - 2, hb)

                @pl.when(hi < n_chunks)
                def _():
                    start_gather(hi, hb)

                drain_gather(b)
                add_pos(b)
                start_wb(ci, b)
            return carry

        lax.fori_loop(0, n_chunks // NBUF, outer, 0)
        drain_wb(n_chunks - 2, (n_chunks - 2) % NBUF)
        drain_wb(n_chunks - 1, (n_chunks - 1) % NBUF)

    return gather_add


def kernel(phrase, phrase_emb_weight, pos_emb_weight):
    B, Lseq = phrase.shape
    rows = B * Lseq
    n_chunks = rows // (NW * CHUNK)
    idx = phrase.reshape(NW, n_chunks, CHUNK // GB, GB).astype(jnp.int32)
    pos = pos_emb_weight[:Lseq]
    out = _build_kernel(n_chunks)(idx, phrase_emb_weight, pos)
    return out.reshape(B, Lseq, phrase_emb_weight.shape[1])


# speedup vs baseline: 7.7697x; 1.1527x over previous
import functools

import jax
import jax.numpy as jnp
from jax import lax
from jax.experimental import pallas as pl
from jax.experimental.pallas import tpu as pltpu
from jax.experimental.pallas import tpu_sc as plsc

D = 64
DP = 128
L = 50
NC = 2
NS = 16
NW = NC * NS
NBUF = 4


def _build_kernel(n_phr):
    mesh = plsc.VectorSubcoreMesh(core_axis_name="c", subcore_axis_name="s")
    n_phrases = NW * n_phr

    @functools.partial(
        pl.kernel,
        mesh=mesh,
        out_type=jax.ShapeDtypeStruct((n_phrases, L, D), jnp.float32),
        scratch_types=(
            [pltpu.VMEM((n_phr, L), jnp.int32),
             pltpu.VMEM((L, D), jnp.float32)]
            + [pltpu.VMEM((L, DP), jnp.float32) for _ in range(NBUF)]
            + [pltpu.VMEM((L, D), jnp.float32) for _ in range(NBUF)]
            + [pltpu.SemaphoreType.DMA for _ in range(2 * NBUF)]
        ),
    )
    def gather_add(idx_hbm, table_hbm, pos_hbm, out_hbm, idx_v, pos_v, *rest):
        bufs = rest[:NBUF]
        stg = rest[NBUF:2 * NBUF]
        gsem = rest[2 * NBUF:3 * NBUF]
        wsem = rest[3 * NBUF:]
        wid = lax.axis_index("s") * NC + lax.axis_index("c")
        pbase = wid * n_phr
        pltpu.sync_copy(idx_hbm.at[wid], idx_v)
        pltpu.sync_copy(pos_hbm, pos_v)

        def start_gather(ci, b):
            pltpu.async_copy(table_hbm.at[idx_v.at[ci]], bufs[b], gsem[b])

        def drain_gather(b):
            pltpu.make_async_copy(
                table_hbm.at[idx_v.at[0]], bufs[b], gsem[b]).wait()

        def add_pos(b):
            for l in range(L):
                for j in range(D // 16):
                    sl = pl.ds(j * 16, 16)
                    stg[b][l, sl] = bufs[b][l, sl] + pos_v[l, sl]

        def start_wb(ci, b):
            pltpu.async_copy(stg[b], out_hbm.at[pbase + ci], wsem[b])

        def drain_wb(ci, b):
            pltpu.make_async_copy(
                stg[b], out_hbm.at[pbase + ci], wsem[b]).wait()

        start_gather(0, 0)
        start_gather(1, 1)

        def outer(go, carry):
            for b in range(NBUF):
                ci = go * NBUF + b
                hb = (b + 2) % NBUF
                hi = ci + 2
                @pl.when(ci >= 2)
                def _():
                    drain_wb(ci - 2, hb)

                @pl.when(hi < n_phr)
                def _():
                    start_gather(hi, hb)

                drain_gather(b)
                add_pos(b)
                start_wb(ci, b)
            return carry

        lax.fori_loop(0, n_phr // NBUF, outer, 0)
        drain_wb(n_phr - 2, (n_phr - 2) % NBUF)
        drain_wb(n_phr - 1, (n_phr - 1) % NBUF)

    return gather_add


def kernel(phrase, phrase_emb_weight, pos_emb_weight):
    B, Lseq = phrase.shape
    n_phr = B // NW
    idx = phrase.reshape(NW, n_phr, Lseq).astype(jnp.int32)
    table = jnp.pad(phrase_emb_weight, ((0, 0), (0, DP - D)))
    pos = pos_emb_weight[:Lseq]
    out = _build_kernel(n_phr)(idx, table, pos)
    return out.reshape(B, Lseq, phrase_emb_weight.shape[1])
